# baseline (speedup 1.0000x reference)
import functools

import jax
import jax.numpy as jnp
from jax import lax
from jax.experimental import pallas as pl
from jax.experimental.pallas import tpu as pltpu
from jax.experimental.pallas import tpu_sc as plsc

N = 10000
E = 320000
EP = 327680
D = 128
NC = 2
NS = 16
NW = NC * NS
K = 128
BPW = EP // NW // K
KG = 64
BPG = EP // NW // KG
QB = BPG // 4
NP = 10240
RPS = NP // NS

_mesh = plsc.VectorSubcoreMesh(core_axis_name="c", subcore_axis_name="s",
                               num_cores=NC, num_subcores=NS)


@functools.partial(
    pl.kernel,
    out_type=jax.ShapeDtypeStruct((NC, NP, D), jnp.float32),
    mesh=_mesh,
    scratch_types=[
        pltpu.VMEM_SHARED((NP, D), jnp.float32),
        pltpu.VMEM((BPW, K), jnp.int32),
        pltpu.VMEM((K, D), jnp.float32),
        pltpu.SemaphoreType.DMA,
    ],
)
def _sc_degree(col_hbm, zeros_hbm, ones_hbm, out_hbm, acc, cidx, ones, dsem):
    cid = lax.axis_index("c")
    sid = lax.axis_index("s")
    wid = sid * NC + cid
    pltpu.sync_copy(zeros_hbm, acc.at[pl.ds(sid * RPS, RPS)])
    pltpu.sync_copy(col_hbm.at[wid], cidx)
    pltpu.sync_copy(ones_hbm, ones)
    plsc.subcore_barrier()

    def body(j, carry):
        pltpu.async_copy(ones, acc.at[cidx.at[j]], dsem, add=True)

        @pl.when(j >= 3)
        def _():
            pltpu.make_async_copy(ones, acc.at[cidx.at[j - 3]], dsem).wait()

        return carry

    lax.fori_loop(0, BPW, body, 0)
    for t in range(3):
        pltpu.make_async_copy(ones, acc.at[cidx.at[BPW - 3 + t]], dsem).wait()
    plsc.subcore_barrier()
    pltpu.sync_copy(acc.at[pl.ds(sid * RPS, RPS)],
                    out_hbm.at[cid].at[pl.ds(sid * RPS, RPS)])


@functools.partial(
    pl.kernel,
    out_type=jax.ShapeDtypeStruct((NC, NP, D), jnp.float32),
    mesh=_mesh,
    scratch_types=[
        pltpu.VMEM_SHARED((NP, D), jnp.float32),
        pltpu.VMEM((QB, KG), jnp.int32),
        pltpu.VMEM((QB, KG), jnp.int32),
        pltpu.VMEM((KG, D), jnp.float32),
        pltpu.VMEM((KG, D), jnp.float32),
        pltpu.VMEM((KG, D), jnp.float32),
        pltpu.VMEM((KG, D), jnp.float32),
        pltpu.SemaphoreType.DMA,
    ],
)
def _sc_aggregate(hs_hbm, row_hbm, col_hbm, zeros_hbm, out_hbm,
                  acc, ridx, cidx, rows0, rows1, rows2, rows3, sem):
    cid = lax.axis_index("c")
    sid = lax.axis_index("s")
    wid = sid * NC + cid
    bufs = (rows0, rows1, rows2, rows3)
    pltpu.sync_copy(zeros_hbm, acc.at[pl.ds(sid * RPS, RPS)])
    plsc.subcore_barrier()

    for h in range(4):
        pltpu.sync_copy(row_hbm.at[wid].at[pl.ds(h * QB, QB)], ridx)
        pltpu.sync_copy(col_hbm.at[wid].at[pl.ds(h * QB, QB)], cidx)
        pltpu.async_copy(hs_hbm.at[ridx.at[0]], rows0, sem)
        pltpu.async_copy(hs_hbm.at[ridx.at[1]], rows1, sem)

        def body(j4, carry):
            for t in range(4):
                j = 4 * j4 + t
                cur = bufs[t]
                nxt = bufs[(t + 2) % 4]
                pltpu.make_async_copy(hs_hbm.at[ridx.at[j]], cur, sem).wait()
                pltpu.async_copy(hs_hbm.at[ridx.at[j + 2]], nxt, sem)
                pltpu.sync_copy(cur, acc.at[cidx.at[j]], add=True)
            return carry

        lax.fori_loop(0, QB // 4 - 1, body, 0)
        for t in range(4):
            j = QB - 4 + t
            cur = bufs[t]
            pltpu.make_async_copy(hs_hbm.at[ridx.at[j]], cur, sem).wait()

            if t < 2:
                pltpu.async_copy(hs_hbm.at[ridx.at[j + 2]], bufs[t + 2], sem)
            pltpu.sync_copy(cur, acc.at[cidx.at[j]], add=True)
    plsc.subcore_barrier()
    pltpu.sync_copy(acc.at[pl.ds(sid * RPS, RPS)],
                    out_hbm.at[cid].at[pl.ds(sid * RPS, RPS)])


RB = 400


def _dinv_block(deg_ref):
    d = deg_ref[0][:, 0:1] + deg_ref[1][:, 0:1]
    return lax.rsqrt(d + 1.0)


def _tc_first(x, W1, deg):
    def body(x_ref, w_ref, deg_ref, o_ref, dinv_ref):
        mm = jnp.dot(x_ref[...], w_ref[...], preferred_element_type=jnp.float32)
        dinv = _dinv_block(deg_ref)
        o_ref[...] = mm * dinv
        dinv_ref[...] = jnp.broadcast_to(dinv, (RB, 8))

    return pl.pallas_call(
        body,
        grid=(N // RB,),
        in_specs=[
            pl.BlockSpec((RB, D), lambda i: (i, 0)),
            pl.BlockSpec((D, D), lambda i: (0, 0)),
            pl.BlockSpec((NC, RB, D), lambda i: (0, i, 0)),
        ],
        out_specs=[
            pl.BlockSpec((RB, D), lambda i: (i, 0)),
            pl.BlockSpec((RB, 8), lambda i: (i, 0)),
        ],
        out_shape=[
            jax.ShapeDtypeStruct((N, D), jnp.float32),
            jax.ShapeDtypeStruct((N, 8), jnp.float32),
        ],
    )(x, W1, deg)


def _tc_mid(A, hs, b, W2, dinv8):
    def body(a_ref, hs_ref, b_ref, w_ref, dinv_ref, o_ref):
        dinv = dinv_ref[:, 0:1]
        h = (a_ref[0] + a_ref[1] + hs_ref[...]) * dinv + b_ref[...]
        h = jnp.maximum(h, 0.0)
        mm = jnp.dot(h, w_ref[...], preferred_element_type=jnp.float32)
        o_ref[...] = mm * dinv

    return pl.pallas_call(
        body,
        grid=(N // RB,),
        in_specs=[
            pl.BlockSpec((NC, RB, D), lambda i: (0, i, 0)),
            pl.BlockSpec((RB, D), lambda i: (i, 0)),
            pl.BlockSpec((1, D), lambda i: (0, 0)),
            pl.BlockSpec((D, D), lambda i: (0, 0)),
            pl.BlockSpec((RB, 8), lambda i: (i, 0)),
        ],
        out_specs=pl.BlockSpec((RB, D), lambda i: (i, 0)),
        out_shape=jax.ShapeDtypeStruct((N, D), jnp.float32),
    )(A, hs, b, W2, dinv8)


def _tc_final(A, hs, b, dinv8):
    def body(a_ref, hs_ref, b_ref, dinv_ref, o_ref):
        dinv = dinv_ref[:, 0:1]
        h = (a_ref[0] + a_ref[1] + hs_ref[...]) * dinv + b_ref[...]
        o_ref[...] = jnp.maximum(h, 0.0)

    return pl.pallas_call(
        body,
        grid=(N // RB,),
        in_specs=[
            pl.BlockSpec((NC, RB, D), lambda i: (0, i, 0)),
            pl.BlockSpec((RB, D), lambda i: (i, 0)),
            pl.BlockSpec((1, D), lambda i: (0, 0)),
            pl.BlockSpec((RB, 8), lambda i: (i, 0)),
        ],
        out_specs=pl.BlockSpec((RB, D), lambda i: (i, 0)),
        out_shape=jax.ShapeDtypeStruct((N, D), jnp.float32),
    )(A, hs, b, dinv8)


def kernel(x, edge_index, W1, b1, W2, b2):
    pad = jnp.arange(EP - E, dtype=jnp.int32)
    rowf = jnp.concatenate([edge_index[0].astype(jnp.int32), pad % N])
    colf = jnp.concatenate([edge_index[1].astype(jnp.int32), N + pad % (NP - N)])
    col = colf.reshape(NW, BPW, K)
    rowg = rowf.reshape(NW, BPG, KG)
    colg = colf.reshape(NW, BPG, KG)
    zeros = jnp.zeros((RPS, D), jnp.float32)
    onesD = jnp.ones((K, D), jnp.float32)
    b1r = b1.reshape(1, D)
    b2r = b2.reshape(1, D)

    deg = _sc_degree(col, zeros, onesD)
    hs1, dinv8 = _tc_first(x, W1, deg)
    A1 = _sc_aggregate(hs1, rowg, colg, zeros)
    hs2 = _tc_mid(A1, hs1, b1r, W2, dinv8)
    A2 = _sc_aggregate(hs2, rowg, colg, zeros)
    return _tc_final(A2, hs2, b2r, dinv8)


# speedup vs baseline: 1.0831x; 1.0831x over previous
import functools

import jax
import jax.numpy as jnp
from jax import lax
from jax.experimental import pallas as pl
from jax.experimental.pallas import tpu as pltpu
from jax.experimental.pallas import tpu_sc as plsc

N = 10000
E = 320000
EP = 327680
D = 128
NC = 2
NS = 16
NW = NC * NS
K = 128
BPW = EP // NW // K
KG = 64
BPG = EP // NW // KG
QB = BPG // 4
NP = 10240
RPS = NP // NS

_mesh = plsc.VectorSubcoreMesh(core_axis_name="c", subcore_axis_name="s",
                               num_cores=NC, num_subcores=NS)


@functools.partial(
    pl.kernel,
    out_type=jax.ShapeDtypeStruct((NC, NP, D), jnp.float32),
    mesh=_mesh,
    scratch_types=[
        pltpu.VMEM_SHARED((NP, D), jnp.float32),
        pltpu.VMEM((BPW, K), jnp.int32),
        pltpu.VMEM((K, D), jnp.float32),
        pltpu.SemaphoreType.DMA,
    ],
)
def _sc_degree(col_hbm, zeros_hbm, ones_hbm, out_hbm, acc, cidx, ones, dsem):
    cid = lax.axis_index("c")
    sid = lax.axis_index("s")
    wid = sid * NC + cid
    pltpu.sync_copy(zeros_hbm.at[pl.ds(sid * RPS, RPS)],
                    acc.at[pl.ds(sid * RPS, RPS)])
    pltpu.sync_copy(col_hbm.at[wid], cidx)
    pltpu.sync_copy(ones_hbm, ones)
    plsc.subcore_barrier()

    def body(j, carry):
        pltpu.async_copy(ones, acc.at[cidx.at[j]], dsem, add=True)

        @pl.when(j >= 3)
        def _():
            pltpu.make_async_copy(ones, acc.at[cidx.at[j - 3]], dsem).wait()

        return carry

    lax.fori_loop(0, BPW, body, 0)
    for t in range(3):
        pltpu.make_async_copy(ones, acc.at[cidx.at[BPW - 3 + t]], dsem).wait()
    plsc.subcore_barrier()
    pltpu.sync_copy(acc.at[pl.ds(sid * RPS, RPS)],
                    out_hbm.at[cid].at[pl.ds(sid * RPS, RPS)])


@functools.partial(
    pl.kernel,
    out_type=jax.ShapeDtypeStruct((NC, NP, D), jnp.float32),
    mesh=_mesh,
    scratch_types=[
        pltpu.VMEM_SHARED((NP, D), jnp.float32),
        pltpu.VMEM((QB, KG), jnp.int32),
        pltpu.VMEM((QB, KG), jnp.int32),
        pltpu.VMEM((KG, D), jnp.float32),
        pltpu.VMEM((KG, D), jnp.float32),
        pltpu.VMEM((KG, D), jnp.float32),
        pltpu.VMEM((KG, D), jnp.float32),
        pltpu.SemaphoreType.DMA,
    ],
)
def _sc_aggregate(hs_hbm, row_hbm, col_hbm, zeros_hbm, out_hbm,
                  acc, ridx, cidx, rows0, rows1, rows2, rows3, sem):
    cid = lax.axis_index("c")
    sid = lax.axis_index("s")
    wid = sid * NC + cid
    bufs = (rows0, rows1, rows2, rows3)
    pltpu.sync_copy(zeros_hbm.at[pl.ds(sid * RPS, RPS)],
                    acc.at[pl.ds(sid * RPS, RPS)])
    plsc.subcore_barrier()

    for h in range(4):
        pltpu.sync_copy(row_hbm.at[wid].at[pl.ds(h * QB, QB)], ridx)
        pltpu.sync_copy(col_hbm.at[wid].at[pl.ds(h * QB, QB)], cidx)
        pltpu.async_copy(hs_hbm.at[ridx.at[0]], rows0, sem)
        pltpu.async_copy(hs_hbm.at[ridx.at[1]], rows1, sem)
        pltpu.async_copy(hs_hbm.at[ridx.at[2]], rows2, sem)

        def body(j4, carry):
            for t in range(4):
                j = 4 * j4 + t
                cur = bufs[t]
                nxt = bufs[(t + 3) % 4]
                pltpu.make_async_copy(hs_hbm.at[ridx.at[j]], cur, sem).wait()
                pltpu.async_copy(hs_hbm.at[ridx.at[j + 3]], nxt, sem)
                pltpu.sync_copy(cur, acc.at[cidx.at[j]], add=True)
            return carry

        lax.fori_loop(0, QB // 4 - 1, body, 0)
        for t in range(4):
            j = QB - 4 + t
            cur = bufs[t]
            pltpu.make_async_copy(hs_hbm.at[ridx.at[j]], cur, sem).wait()

            if t < 1:
                pltpu.async_copy(hs_hbm.at[ridx.at[j + 3]], bufs[(t + 3) % 4], sem)
            pltpu.sync_copy(cur, acc.at[cidx.at[j]], add=True)
    plsc.subcore_barrier()
    pltpu.sync_copy(acc.at[pl.ds(sid * RPS, RPS)],
                    out_hbm.at[cid].at[pl.ds(sid * RPS, RPS)])


RB = 400


def _dinv_block(deg_ref):
    d = deg_ref[0][:, 0:1] + deg_ref[1][:, 0:1]
    return lax.rsqrt(d + 1.0)


def _tc_first(x, W1, deg):
    def body(x_ref, w_ref, deg_ref, o_ref, dinv_ref):
        mm = jnp.dot(x_ref[...], w_ref[...], preferred_element_type=jnp.float32)
        dinv = _dinv_block(deg_ref)
        o_ref[...] = mm * dinv
        dinv_ref[...] = jnp.broadcast_to(dinv, (RB, 8))

    return pl.pallas_call(
        body,
        grid=(N // RB,),
        in_specs=[
            pl.BlockSpec((RB, D), lambda i: (i, 0)),
            pl.BlockSpec((D, D), lambda i: (0, 0)),
            pl.BlockSpec((NC, RB, D), lambda i: (0, i, 0)),
        ],
        out_specs=[
            pl.BlockSpec((RB, D), lambda i: (i, 0)),
            pl.BlockSpec((RB, 8), lambda i: (i, 0)),
        ],
        out_shape=[
            jax.ShapeDtypeStruct((N, D), jnp.float32),
            jax.ShapeDtypeStruct((N, 8), jnp.float32),
        ],
    )(x, W1, deg)


def _tc_mid(A, hs, b, W2, dinv8):
    def body(a_ref, hs_ref, b_ref, w_ref, dinv_ref, o_ref):
        dinv = dinv_ref[:, 0:1]
        h = (a_ref[0] + a_ref[1] + hs_ref[...]) * dinv + b_ref[...]
        h = jnp.maximum(h, 0.0)
        mm = jnp.dot(h, w_ref[...], preferred_element_type=jnp.float32)
        o_ref[...] = mm * dinv

    return pl.pallas_call(
        body,
        grid=(N // RB,),
        in_specs=[
            pl.BlockSpec((NC, RB, D), lambda i: (0, i, 0)),
            pl.BlockSpec((RB, D), lambda i: (i, 0)),
            pl.BlockSpec((1, D), lambda i: (0, 0)),
            pl.BlockSpec((D, D), lambda i: (0, 0)),
            pl.BlockSpec((RB, 8), lambda i: (i, 0)),
        ],
        out_specs=pl.BlockSpec((RB, D), lambda i: (i, 0)),
        out_shape=jax.ShapeDtypeStruct((N, D), jnp.float32),
    )(A, hs, b, W2, dinv8)


def _tc_final(A, hs, b, dinv8):
    def body(a_ref, hs_ref, b_ref, dinv_ref, o_ref):
        dinv = dinv_ref[:, 0:1]
        h = (a_ref[0] + a_ref[1] + hs_ref[...]) * dinv + b_ref[...]
        o_ref[...] = jnp.maximum(h, 0.0)

    return pl.pallas_call(
        body,
        grid=(N // RB,),
        in_specs=[
            pl.BlockSpec((NC, RB, D), lambda i: (0, i, 0)),
            pl.BlockSpec((RB, D), lambda i: (i, 0)),
            pl.BlockSpec((1, D), lambda i: (0, 0)),
            pl.BlockSpec((RB, 8), lambda i: (i, 0)),
        ],
        out_specs=pl.BlockSpec((RB, D), lambda i: (i, 0)),
        out_shape=jax.ShapeDtypeStruct((N, D), jnp.float32),
    )(A, hs, b, dinv8)


def kernel(x, edge_index, W1, b1, W2, b2):
    pad = jnp.arange(EP - E, dtype=jnp.int32)
    rowf = jnp.concatenate([edge_index[0].astype(jnp.int32), pad % N])
    colf = jnp.concatenate([edge_index[1].astype(jnp.int32), N + pad % (NP - N)])
    col = colf.reshape(NW, BPW, K)
    rowg = rowf.reshape(NW, BPG, KG)
    colg = colf.reshape(NW, BPG, KG)
    zeros = jnp.zeros((NP, D), jnp.float32)
    onesD = jnp.ones((K, D), jnp.float32)
    b1r = b1.reshape(1, D)
    b2r = b2.reshape(1, D)

    deg = _sc_degree(col, zeros, onesD)
    hs1, dinv8 = _tc_first(x, W1, deg)
    A1 = _sc_aggregate(hs1, rowg, colg, zeros)
    hs2 = _tc_mid(A1, hs1, b1r, W2, dinv8)
    A2 = _sc_aggregate(hs2, rowg, colg, zeros)
    return _tc_final(A2, hs2, b2r, dinv8)


# speedup vs baseline: 1.0836x; 1.0004x over previous
import functools

import jax
import jax.numpy as jnp
from jax import lax
from jax.experimental import pallas as pl
from jax.experimental.pallas import tpu as pltpu
from jax.experimental.pallas import tpu_sc as plsc

N = 10000
E = 320000
EP = 327680
D = 128
NC = 2
NS = 16
NW = NC * NS
K = 128
BPW = EP // NW // K
KG = 64
BPG = EP // NW // KG
QB = BPG // 4
NP = 10240
RPS = NP // NS

_mesh = plsc.VectorSubcoreMesh(core_axis_name="c", subcore_axis_name="s",
                               num_cores=NC, num_subcores=NS)


@functools.partial(
    pl.kernel,
    out_type=jax.ShapeDtypeStruct((NC, NP, D), jnp.float32),
    mesh=_mesh,
    scratch_types=[
        pltpu.VMEM_SHARED((NP, D), jnp.float32),
        pltpu.VMEM((BPW, K), jnp.int32),
        pltpu.VMEM((K, D), jnp.float32),
        pltpu.SemaphoreType.DMA,
    ],
)
def _sc_degree(col_hbm, zeros_hbm, ones_hbm, out_hbm, acc, cidx, ones, dsem):
    cid = lax.axis_index("c")
    sid = lax.axis_index("s")
    wid = sid * NC + cid
    pltpu.sync_copy(zeros_hbm.at[pl.ds(sid * RPS, RPS)],
                    acc.at[pl.ds(sid * RPS, RPS)])
    pltpu.sync_copy(col_hbm.at[wid], cidx)
    pltpu.sync_copy(ones_hbm, ones)
    plsc.subcore_barrier()

    def body(j, carry):
        pltpu.async_copy(ones, acc.at[cidx.at[j]], dsem, add=True)

        @pl.when(j >= 3)
        def _():
            pltpu.make_async_copy(ones, acc.at[cidx.at[j - 3]], dsem).wait()

        return carry

    lax.fori_loop(0, BPW, body, 0)
    for t in range(3):
        pltpu.make_async_copy(ones, acc.at[cidx.at[BPW - 3 + t]], dsem).wait()
    plsc.subcore_barrier()
    pltpu.sync_copy(acc.at[pl.ds(sid * RPS, RPS)],
                    out_hbm.at[cid].at[pl.ds(sid * RPS, RPS)])


@functools.partial(
    pl.kernel,
    out_type=jax.ShapeDtypeStruct((NC, NP, D), jnp.float32),
    mesh=_mesh,
    scratch_types=[
        pltpu.VMEM_SHARED((NP, D), jnp.float32),
        pltpu.VMEM((QB, KG), jnp.int32),
        pltpu.VMEM((QB, KG), jnp.int32),
        pltpu.VMEM((KG, D), jnp.float32),
        pltpu.VMEM((KG, D), jnp.float32),
        pltpu.VMEM((KG, D), jnp.float32),
        pltpu.VMEM((KG, D), jnp.float32),
        pltpu.SemaphoreType.DMA,
        pltpu.SemaphoreType.DMA,
    ],
)
def _sc_aggregate(hs_hbm, row_hbm, col_hbm, zeros_hbm, out_hbm,
                  acc, ridx, cidx, rows0, rows1, rows2, rows3, sem, ssem):
    cid = lax.axis_index("c")
    sid = lax.axis_index("s")
    wid = sid * NC + cid
    bufs = (rows0, rows1, rows2, rows3)
    pltpu.sync_copy(zeros_hbm.at[pl.ds(sid * RPS, RPS)],
                    acc.at[pl.ds(sid * RPS, RPS)])
    plsc.subcore_barrier()

    for h in range(4):
        pltpu.sync_copy(row_hbm.at[wid].at[pl.ds(h * QB, QB)], ridx)
        pltpu.sync_copy(col_hbm.at[wid].at[pl.ds(h * QB, QB)], cidx)
        pltpu.async_copy(hs_hbm.at[ridx.at[0]], rows0, sem)
        pltpu.async_copy(hs_hbm.at[ridx.at[1]], rows1, sem)
        pltpu.async_copy(hs_hbm.at[ridx.at[2]], rows2, sem)
        pltpu.make_async_copy(hs_hbm.at[ridx.at[0]], rows0, sem).wait()
        pltpu.async_copy(hs_hbm.at[ridx.at[3]], rows3, sem)
        pltpu.async_copy(rows0, acc.at[cidx.at[0]], ssem, add=True)

        def body(j4, carry):
            for t in range(4):
                j = 4 * j4 + 1 + t
                cur = bufs[(1 + t) % 4]
                prv = bufs[t]
                pltpu.make_async_copy(hs_hbm.at[ridx.at[j]], cur, sem).wait()
                pltpu.make_async_copy(prv, acc.at[cidx.at[j - 1]], ssem).wait()
                pltpu.async_copy(hs_hbm.at[ridx.at[j + 3]], prv, sem)
                pltpu.async_copy(cur, acc.at[cidx.at[j]], ssem, add=True)
            return carry

        lax.fori_loop(0, QB // 4 - 1, body, 0)
        for t in range(3):
            j = QB - 3 + t
            cur = bufs[(1 + t) % 4]
            prv = bufs[t]
            pltpu.make_async_copy(hs_hbm.at[ridx.at[j]], cur, sem).wait()
            pltpu.make_async_copy(prv, acc.at[cidx.at[j - 1]], ssem).wait()
            pltpu.async_copy(cur, acc.at[cidx.at[j]], ssem, add=True)
        pltpu.make_async_copy(bufs[3], acc.at[cidx.at[QB - 1]], ssem).wait()
    plsc.subcore_barrier()
    pltpu.sync_copy(acc.at[pl.ds(sid * RPS, RPS)],
                    out_hbm.at[cid].at[pl.ds(sid * RPS, RPS)])


RB = 400


def _dinv_block(deg_ref):
    d = deg_ref[0][:, 0:1] + deg_ref[1][:, 0:1]
    return lax.rsqrt(d + 1.0)


def _tc_first(x, W1, deg):
    def body(x_ref, w_ref, deg_ref, o_ref, dinv_ref):
        mm = jnp.dot(x_ref[...], w_ref[...], preferred_element_type=jnp.float32)
        dinv = _dinv_block(deg_ref)
        o_ref[...] = mm * dinv
        dinv_ref[...] = jnp.broadcast_to(dinv, (RB, 8))

    return pl.pallas_call(
        body,
        grid=(N // RB,),
        in_specs=[
            pl.BlockSpec((RB, D), lambda i: (i, 0)),
            pl.BlockSpec((D, D), lambda i: (0, 0)),
            pl.BlockSpec((NC, RB, D), lambda i: (0, i, 0)),
        ],
        out_specs=[
            pl.BlockSpec((RB, D), lambda i: (i, 0)),
            pl.BlockSpec((RB, 8), lambda i: (i, 0)),
        ],
        out_shape=[
            jax.ShapeDtypeStruct((N, D), jnp.float32),
            jax.ShapeDtypeStruct((N, 8), jnp.float32),
        ],
    )(x, W1, deg)


def _tc_mid(A, hs, b, W2, dinv8):
    def body(a_ref, hs_ref, b_ref, w_ref, dinv_ref, o_ref):
        dinv = dinv_ref[:, 0:1]
        h = (a_ref[0] + a_ref[1] + hs_ref[...]) * dinv + b_ref[...]
        h = jnp.maximum(h, 0.0)
        mm = jnp.dot(h, w_ref[...], preferred_element_type=jnp.float32)
        o_ref[...] = mm * dinv

    return pl.pallas_call(
        body,
        grid=(N // RB,),
        in_specs=[
            pl.BlockSpec((NC, RB, D), lambda i: (0, i, 0)),
            pl.BlockSpec((RB, D), lambda i: (i, 0)),
            pl.BlockSpec((1, D), lambda i: (0, 0)),
            pl.BlockSpec((D, D), lambda i: (0, 0)),
            pl.BlockSpec((RB, 8), lambda i: (i, 0)),
        ],
        out_specs=pl.BlockSpec((RB, D), lambda i: (i, 0)),
        out_shape=jax.ShapeDtypeStruct((N, D), jnp.float32),
    )(A, hs, b, W2, dinv8)


def _tc_final(A, hs, b, dinv8):
    def body(a_ref, hs_ref, b_ref, dinv_ref, o_ref):
        dinv = dinv_ref[:, 0:1]
        h = (a_ref[0] + a_ref[1] + hs_ref[...]) * dinv + b_ref[...]
        o_ref[...] = jnp.maximum(h, 0.0)

    return pl.pallas_call(
        body,
        grid=(N // RB,),
        in_specs=[
            pl.BlockSpec((NC, RB, D), lambda i: (0, i, 0)),
            pl.BlockSpec((RB, D), lambda i: (i, 0)),
            pl.BlockSpec((1, D), lambda i: (0, 0)),
            pl.BlockSpec((RB, 8), lambda i: (i, 0)),
        ],
        out_specs=pl.BlockSpec((RB, D), lambda i: (i, 0)),
        out_shape=jax.ShapeDtypeStruct((N, D), jnp.float32),
    )(A, hs, b, dinv8)


def kernel(x, edge_index, W1, b1, W2, b2):
    pad = jnp.arange(EP - E, dtype=jnp.int32)
    rowf = jnp.concatenate([edge_index[0].astype(jnp.int32), pad % N])
    colf = jnp.concatenate([edge_index[1].astype(jnp.int32), N + pad % (NP - N)])
    col = colf.reshape(NW, BPW, K)
    rowg = rowf.reshape(NW, BPG, KG)
    colg = colf.reshape(NW, BPG, KG)
    zeros = jnp.zeros((NP, D), jnp.float32)
    onesD = jnp.ones((K, D), jnp.float32)
    b1r = b1.reshape(1, D)
    b2r = b2.reshape(1, D)

    deg = _sc_degree(col, zeros, onesD)
    hs1, dinv8 = _tc_first(x, W1, deg)
    A1 = _sc_aggregate(hs1, rowg, colg, zeros)
    hs2 = _tc_mid(A1, hs1, b1r, W2, dinv8)
    A2 = _sc_aggregate(hs2, rowg, colg, zeros)
    return _tc_final(A2, hs2, b2r, dinv8)
